# baseline (speedup 1.0000x reference)
import functools

import jax
import jax.numpy as jnp
from jax.experimental import pallas as pl
from jax.experimental.pallas import tpu as pltpu
from jax.experimental.pallas import tpu_sc as plsc

BLK_R = 16
LANES = 16


def kernel(x, pos_table):
    batch, seq, d_model = x.shape
    xf = x.reshape(batch * seq, d_model)
    n_sblk = seq // BLK_R
    mesh = plsc.VectorSubcoreMesh(core_axis_name="c", subcore_axis_name="s")

    @functools.partial(
        pl.kernel,
        mesh=mesh,
        out_type=jax.ShapeDtypeStruct((batch * seq, d_model), jnp.float32),
    )
    def sc_k(x_hbm, pos_hbm, o_hbm):
        def body(x_vmem, p_vmem, o_vmem):
            @pl.loop(0, BLK_R)
            def _row(r):
                @pl.loop(0, d_model, step=8 * LANES)
                def _col(c):
                    for j in range(8):
                        sl = pl.ds(c + j * LANES, LANES)
                        o_vmem[r, sl] = x_vmem[r, sl] + p_vmem[r, sl]

        pltpu.emit_pipeline(
            body,
            grid=(batch, n_sblk),
            in_specs=[
                pl.BlockSpec((BLK_R, d_model), lambda b, i: (b * n_sblk + i, 0)),
                pl.BlockSpec((BLK_R, d_model), lambda b, i: (i, 0)),
            ],
            out_specs=[
                pl.BlockSpec((BLK_R, d_model), lambda b, i: (b * n_sblk + i, 0)),
            ],
            core_axis_name=("c", "s"),
            dimension_semantics=(pltpu.PARALLEL, pltpu.PARALLEL),
        )(x_hbm, pos_hbm, o_hbm)

    return sc_k(xf, pos_table).reshape(batch, seq, d_model)


# speedup vs baseline: 3.8497x; 3.8497x over previous
import jax
import jax.numpy as jnp
from jax.experimental import pallas as pl


def _add_pos_kernel(x_ref, pos_ref, out_ref):
    out_ref[...] = x_ref[...] + pos_ref[...][None, :, :]


def kernel(x, pos_table):
    batch, seq, d_model = x.shape
    bs = 512
    grid = (seq // bs, batch)
    return pl.pallas_call(
        _add_pos_kernel,
        grid=grid,
        in_specs=[
            pl.BlockSpec((1, bs, d_model), lambda i, b: (b, i, 0)),
            pl.BlockSpec((bs, d_model), lambda i, b: (i, 0)),
        ],
        out_specs=pl.BlockSpec((1, bs, d_model), lambda i, b: (b, i, 0)),
        out_shape=jax.ShapeDtypeStruct((batch, seq, d_model), x.dtype),
    )(x, pos_table[:seq])


# speedup vs baseline: 4.4803x; 1.1638x over previous
import jax
import jax.numpy as jnp
from jax.experimental import pallas as pl


def _add_pos_kernel(x_ref, pos_ref, out_ref):
    out_ref[...] = x_ref[...] + pos_ref[...][None, :, :]


def kernel(x, pos_table):
    batch, seq, d_model = x.shape
    bs = 512
    grid = (seq // bs,)
    return pl.pallas_call(
        _add_pos_kernel,
        grid=grid,
        in_specs=[
            pl.BlockSpec((batch, bs, d_model), lambda i: (0, i, 0)),
            pl.BlockSpec((bs, d_model), lambda i: (i, 0)),
        ],
        out_specs=pl.BlockSpec((batch, bs, d_model), lambda i: (0, i, 0)),
        out_shape=jax.ShapeDtypeStruct((batch, seq, d_model), x.dtype),
    )(x, pos_table[:seq])


# speedup vs baseline: 4.4892x; 1.0020x over previous
import jax
import jax.numpy as jnp
from jax.experimental import pallas as pl
from jax.experimental.pallas import tpu as pltpu


def _add_pos_kernel(x_ref, pos_ref, out_ref):
    out_ref[...] = x_ref[...] + pos_ref[...][None, :, :]


def kernel(x, pos_table):
    batch, seq, d_model = x.shape
    bs = 512
    grid = (seq // bs,)
    return pl.pallas_call(
        _add_pos_kernel,
        grid=grid,
        in_specs=[
            pl.BlockSpec((batch, bs, d_model), lambda i: (0, i, 0)),
            pl.BlockSpec((bs, d_model), lambda i: (i, 0)),
        ],
        out_specs=pl.BlockSpec((batch, bs, d_model), lambda i: (0, i, 0)),
        out_shape=jax.ShapeDtypeStruct((batch, seq, d_model), x.dtype),
        compiler_params=pltpu.CompilerParams(
            dimension_semantics=("parallel",),
        ),
    )(x, pos_table[:seq])
